# baseline (speedup 1.0000x reference)
import functools

import jax
import jax.numpy as jnp
from jax import lax
from jax.experimental import pallas as pl
from jax.experimental.pallas import tpu as pltpu
from jax.experimental.pallas import tpu_sc as plsc

N_NODES = 10000
N_FEATURE = 128
N_HIDDEN = 128
N_CLASS = 64
N_SAMPLE = 32
BATCH = 1024

NW = 32
NODES_PAD = 10240
NPT = NODES_PAD // NW
EPT = NPT * N_SAMPLE
CHUNK_NODES = 4
CHUNK_EDGES = CHUNK_NODES * N_SAMPLE
N_CHUNKS = NPT // CHUNK_NODES
BPT = BATCH // NW
_NBUF = 4

_MESH = plsc.VectorSubcoreMesh(core_axis_name="c", subcore_axis_name="s")


def _accum32(load):
    n_cc = N_FEATURE // 16
    accs = [None] * n_cc
    for j in range(N_SAMPLE):
        for cc in range(n_cc):
            v = load(j, cc)
            accs[cc] = v if j == 0 else accs[cc] + v
    return accs


N_FAST = 320
N_SLOW = 320
FAST_CID = 0
CH_F = N_FAST // CHUNK_NODES
CH_S = N_SLOW // CHUNK_NODES
_NBUF_A = 4
MAXG = CH_F // _NBUF_A
EPT_MAX = N_FAST * N_SAMPLE
IDX_SLACK = EPT_MAX


@functools.partial(
    pl.kernel,
    out_type=jax.ShapeDtypeStruct((NODES_PAD, N_FEATURE), jnp.float32),
    mesh=_MESH,
    scratch_types=[
        pltpu.VMEM((EPT_MAX,), jnp.int32),
        pltpu.VMEM((_NBUF_A, CHUNK_EDGES, N_FEATURE), jnp.float32),
        pltpu.VMEM((_NBUF_A // 2, 2 * CHUNK_NODES, N_FEATURE), jnp.float32),
        [pltpu.SemaphoreType.DMA] * _NBUF_A,
        [pltpu.SemaphoreType.DMA] * (_NBUF_A // 2),
    ],
    compiler_params=pltpu.CompilerParams(needs_layout_passes=False),
)
def _agg1_sum(feature_hbm, nbr_hbm, out_hbm, idx_v, rows_v, out_v, sems,
              sems_o):
    cid = lax.axis_index("c")
    sid = lax.axis_index("s")
    is_fast = cid == FAST_CID
    node_base = jnp.where(is_fast, sid * N_FAST, 16 * N_FAST + sid * N_SLOW)
    n_chunks = jnp.where(is_fast, CH_F, CH_S)

    @pl.when(is_fast)
    def _():
        pltpu.sync_copy(nbr_hbm.at[pl.ds(node_base * N_SAMPLE, EPT_MAX)],
                        idx_v)

    @pl.when(jnp.logical_not(is_fast))
    def _():
        pltpu.sync_copy(
            nbr_hbm.at[pl.ds(node_base * N_SAMPLE, N_SLOW * N_SAMPLE)],
            idx_v.at[pl.ds(0, N_SLOW * N_SAMPLE)])

    def issue(ch, p):
        idx = idx_v.at[pl.ds(ch * CHUNK_EDGES, CHUNK_EDGES)]
        pltpu.async_copy(feature_hbm.at[idx], rows_v.at[p], sems[p])

    def wait(p):
        pltpu.make_async_copy(
            feature_hbm.at[pl.ds(0, CHUNK_EDGES)], rows_v.at[p], sems[p]).wait()

    def wait_store(slot):
        pltpu.make_async_copy(
            out_v.at[slot], out_hbm.at[pl.ds(0, 2 * CHUNK_NODES)],
            sems_o[slot]).wait()

    for p in range(_NBUF_A):
        issue(p, p)

    def chunk_group(g, carry):
        for p in range(_NBUF_A):
            ch = g * _NBUF_A + p
            slot = p >> 1
            half = p & 1

            @pl.when(ch < n_chunks)
            def _(p=p, ch=ch, slot=slot, half=half):
                wait(p)
                if half == 0:
                    @pl.when(g > 0)
                    def _():
                        wait_store(slot)

                def node_body(k, c2, p=p):
                    accs = _accum32(
                        lambda j, cc: rows_v[p, k * N_SAMPLE + j,
                                             pl.ds(cc * 16, 16)])
                    for cc in range(N_FEATURE // 16):
                        out_v[slot, half * CHUNK_NODES + k,
                              pl.ds(cc * 16, 16)] = accs[cc]
                    return c2

                lax.fori_loop(0, CHUNK_NODES, node_body, 0)
                if half == 1:
                    pair_node = node_base + (ch - 1) * CHUNK_NODES
                    pltpu.async_copy(
                        out_v.at[slot],
                        out_hbm.at[pl.ds(pair_node, 2 * CHUNK_NODES)],
                        sems_o[slot])
                @pl.when(ch + _NBUF_A < n_chunks)
                def _():
                    issue(ch + _NBUF_A, p)
        return carry

    lax.fori_loop(0, MAXG, chunk_group, 0)
    for s in range(_NBUF_A // 2):
        wait_store(s)


@functools.partial(
    pl.kernel,
    out_type=(
        jax.ShapeDtypeStruct((BATCH, N_HIDDEN), jnp.float32),
        jax.ShapeDtypeStruct((BATCH, N_HIDDEN), jnp.float32),
    ),
    mesh=_MESH,
    scratch_types=[
        pltpu.VMEM((BPT,), jnp.int32),
        pltpu.VMEM((BPT,), jnp.int32),
        pltpu.VMEM((BPT, 128), jnp.int32),
        pltpu.VMEM((_NBUF, N_SAMPLE), jnp.int32),
        pltpu.VMEM((_NBUF, N_SAMPLE, N_HIDDEN), jnp.float32),
        pltpu.VMEM((BPT, N_HIDDEN), jnp.float32),
        pltpu.VMEM((BPT, N_HIDDEN), jnp.float32),
        [pltpu.SemaphoreType.DMA] * _NBUF,
        pltpu.SemaphoreType.DMA,
    ],
    compiler_params=pltpu.CompilerParams(needs_layout_passes=False),
)
def _hop2(emb1_hbm, nbr4_hbm, nodes_hbm, self_hbm, agg_hbm,
          nodes_v, rows4_v, nlrows_v, nl32_v, rows_v, self_v, out_v,
          sems, sem_s):
    wid = lax.axis_index("s") * 2 + lax.axis_index("c")
    base = wid * BPT
    pltpu.sync_copy(nodes_hbm.at[pl.ds(base, BPT)], nodes_v)
    for i in range(BPT // 16):
        sl = pl.ds(i * 16, 16)
        rows4_v[sl] = jax.lax.shift_right_logical(nodes_v[sl], 2)
    cp_self = pltpu.async_copy(emb1_hbm.at[nodes_v], self_v, sem_s)
    pltpu.async_copy(nbr4_hbm.at[rows4_v], nlrows_v, sems[0]).wait()

    def issue(k, p):
        kf = jnp.full((16,), k, dtype=jnp.int32)
        nv16 = nodes_v[pl.ds((k >> 4) * 16, 16)]
        node_b = lax.gather(
            nv16, jnp.full((16, 1), k & 15, dtype=jnp.int32),
            lax.GatherDimensionNumbers(offset_dims=(), collapsed_slice_dims=(0,),
                                       start_index_map=(0,)),
            (1,), mode=lax.GatherScatterMode.PROMISE_IN_BOUNDS)
        coloff = (node_b & 3) * N_SAMPLE + lax.iota(jnp.int32, 16)
        nl32_v[p, pl.ds(0, 16)] = plsc.load_gather(nlrows_v, [kf, coloff])
        nl32_v[p, pl.ds(16, 16)] = plsc.load_gather(nlrows_v, [kf, coloff + 16])
        pltpu.async_copy(emb1_hbm.at[nl32_v.at[p]], rows_v.at[p], sems[p])

    def wait(p):
        pltpu.make_async_copy(
            emb1_hbm.at[pl.ds(0, N_SAMPLE)], rows_v.at[p], sems[p]).wait()

    for p in range(_NBUF):
        issue(p, p)

    def seed_group(g, carry):
        for p in range(_NBUF):
            k = g * _NBUF + p
            wait(p)
            accs = _accum32(lambda j, cc: rows_v[p, j, pl.ds(cc * 16, 16)])
            for cc in range(N_HIDDEN // 16):
                out_v[k, pl.ds(cc * 16, 16)] = accs[cc]
            @pl.when(k + _NBUF < BPT)
            def _():
                issue(k + _NBUF, p)
        return carry

    lax.fori_loop(0, BPT // _NBUF, seed_group, 0)
    cp_self.wait()
    pltpu.sync_copy(self_v, self_hbm.at[pl.ds(base, BPT)])
    pltpu.sync_copy(out_v, agg_hbm.at[pl.ds(base, BPT)])


def _l1_body(x_ref, a_ref, w_ref, o_ref):
    y = jnp.dot(x_ref[...], w_ref[0:N_FEATURE, :],
                preferred_element_type=jnp.float32)
    y = y + jnp.dot(a_ref[...] * (1.0 / N_SAMPLE), w_ref[N_FEATURE:, :],
                    preferred_element_type=jnp.float32)
    o_ref[...] = jnp.maximum(y, 0.0)


_L1_BLOCK = 1000


def _layer1(feature, aggsum, W1):
    return pl.pallas_call(
        _l1_body,
        grid=(N_NODES // _L1_BLOCK,),
        in_specs=[
            pl.BlockSpec((_L1_BLOCK, N_FEATURE), lambda i: (i, 0)),
            pl.BlockSpec((_L1_BLOCK, N_FEATURE), lambda i: (i, 0)),
            pl.BlockSpec((2 * N_FEATURE, N_HIDDEN), lambda i: (0, 0)),
        ],
        out_specs=pl.BlockSpec((_L1_BLOCK, N_HIDDEN), lambda i: (i, 0)),
        out_shape=jax.ShapeDtypeStruct((N_NODES, N_HIDDEN), jnp.float32),
    )(feature, aggsum, W1)


def _l2_body(s_ref, a_ref, w2_ref, fw_ref, fb_ref, o_ref):
    x2a = s_ref[...]
    x2b = a_ref[...] * (1.0 / N_SAMPLE)
    h = jnp.dot(x2a, w2_ref[0:N_HIDDEN, :], preferred_element_type=jnp.float32)
    h = h + jnp.dot(x2b, w2_ref[N_HIDDEN:, :],
                    preferred_element_type=jnp.float32)
    h = jnp.maximum(h, 0.0)
    s = jnp.dot(h, fw_ref[...], preferred_element_type=jnp.float32)
    s = s + fb_ref[...]
    m = jnp.max(s, axis=1, keepdims=True)
    lse = jnp.log(jnp.sum(jnp.exp(s - m), axis=1, keepdims=True)) + m
    o_ref[...] = s - lse


def _layer2(selfrows, agg2sum, W2, fc_W, fc_b):
    return pl.pallas_call(
        _l2_body,
        grid=(1,),
        in_specs=[
            pl.BlockSpec((BATCH, N_HIDDEN), lambda i: (0, 0)),
            pl.BlockSpec((BATCH, N_HIDDEN), lambda i: (0, 0)),
            pl.BlockSpec((2 * N_HIDDEN, N_HIDDEN), lambda i: (0, 0)),
            pl.BlockSpec((N_HIDDEN, N_CLASS), lambda i: (0, 0)),
            pl.BlockSpec((1, N_CLASS), lambda i: (0, 0)),
        ],
        out_specs=pl.BlockSpec((BATCH, N_CLASS), lambda i: (0, 0)),
        out_shape=jax.ShapeDtypeStruct((BATCH, N_CLASS), jnp.float32),
    )(selfrows, agg2sum, W2, fc_W, fc_b)


def kernel(nodes, feature, neighbor_list, W1, W2, fc_W, fc_b):
    pad_edges = (NODES_PAD - N_NODES) * N_SAMPLE + IDX_SLACK
    nbr_flat = jnp.concatenate(
        [neighbor_list.reshape(-1),
         jnp.zeros((pad_edges,), dtype=jnp.int32)])
    aggsum = _agg1_sum(feature, nbr_flat)
    emb1 = _layer1(feature, aggsum[:N_NODES], W1)
    nbr4 = neighbor_list.reshape(N_NODES * N_SAMPLE // 128, 128)
    selfrows, agg2sum = _hop2(emb1, nbr4, nodes)
    return _layer2(selfrows, agg2sum, W2, fc_W, fc_b.reshape(1, N_CLASS))


# speedup vs baseline: 1.0397x; 1.0397x over previous
import functools

import jax
import jax.numpy as jnp
from jax import lax
from jax.experimental import pallas as pl
from jax.experimental.pallas import tpu as pltpu
from jax.experimental.pallas import tpu_sc as plsc

N_NODES = 10000
N_FEATURE = 128
N_HIDDEN = 128
N_CLASS = 64
N_SAMPLE = 32
BATCH = 1024

NW = 32
NODES_PAD = 10240
NPT = NODES_PAD // NW
EPT = NPT * N_SAMPLE
CHUNK_NODES = 4
CHUNK_EDGES = CHUNK_NODES * N_SAMPLE
N_CHUNKS = NPT // CHUNK_NODES
BPT = BATCH // NW
_NBUF = 4

_MESH = plsc.VectorSubcoreMesh(core_axis_name="c", subcore_axis_name="s")


def _accum32(load):
    n_cc = N_FEATURE // 16
    accs = [None] * n_cc
    for j in range(N_SAMPLE):
        for cc in range(n_cc):
            v = load(j, cc)
            accs[cc] = v if j == 0 else accs[cc] + v
    return accs


N_FAST = 576
N_SLOW = 64
FAST_CID = 0
CH_F = N_FAST // CHUNK_NODES
CH_S = N_SLOW // CHUNK_NODES
_NBUF_A = 6
MAXG = CH_F // _NBUF_A
EPT_MAX = N_FAST * N_SAMPLE
IDX_SLACK = EPT_MAX


@functools.partial(
    pl.kernel,
    out_type=jax.ShapeDtypeStruct((NODES_PAD, N_FEATURE), jnp.float32),
    mesh=_MESH,
    scratch_types=[
        pltpu.VMEM((EPT_MAX,), jnp.int32),
        pltpu.VMEM((_NBUF_A, CHUNK_EDGES, N_FEATURE), jnp.float32),
        pltpu.VMEM((_NBUF_A // 2, 2 * CHUNK_NODES, N_FEATURE), jnp.float32),
        [pltpu.SemaphoreType.DMA] * _NBUF_A,
        [pltpu.SemaphoreType.DMA] * (_NBUF_A // 2),
    ],
    compiler_params=pltpu.CompilerParams(needs_layout_passes=False),
)
def _agg1_sum(feature_hbm, nbr_hbm, out_hbm, idx_v, rows_v, out_v, sems,
              sems_o):
    cid = lax.axis_index("c")
    sid = lax.axis_index("s")
    is_fast = cid == FAST_CID
    node_base = jnp.where(is_fast, sid * N_FAST, 16 * N_FAST + sid * N_SLOW)
    n_chunks = jnp.where(is_fast, CH_F, CH_S)

    @pl.when(is_fast)
    def _():
        pltpu.sync_copy(nbr_hbm.at[pl.ds(node_base * N_SAMPLE, EPT_MAX)],
                        idx_v)

    @pl.when(jnp.logical_not(is_fast))
    def _():
        pltpu.sync_copy(
            nbr_hbm.at[pl.ds(node_base * N_SAMPLE, N_SLOW * N_SAMPLE)],
            idx_v.at[pl.ds(0, N_SLOW * N_SAMPLE)])

    def issue(ch, p):
        idx = idx_v.at[pl.ds(ch * CHUNK_EDGES, CHUNK_EDGES)]
        pltpu.async_copy(feature_hbm.at[idx], rows_v.at[p], sems[p])

    def wait(p):
        pltpu.make_async_copy(
            feature_hbm.at[pl.ds(0, CHUNK_EDGES)], rows_v.at[p], sems[p]).wait()

    def wait_store(slot):
        pltpu.make_async_copy(
            out_v.at[slot], out_hbm.at[pl.ds(0, 2 * CHUNK_NODES)],
            sems_o[slot]).wait()

    for p in range(_NBUF_A):
        issue(p, p)

    def chunk_group(g, carry):
        for p in range(_NBUF_A):
            ch = g * _NBUF_A + p
            slot = p >> 1
            half = p & 1

            @pl.when(ch < n_chunks)
            def _(p=p, ch=ch, slot=slot, half=half):
                wait(p)
                if half == 0:
                    @pl.when(g > 0)
                    def _():
                        wait_store(slot)

                def node_body(k, c2, p=p):
                    accs = _accum32(
                        lambda j, cc: rows_v[p, k * N_SAMPLE + j,
                                             pl.ds(cc * 16, 16)])
                    for cc in range(N_FEATURE // 16):
                        out_v[slot, half * CHUNK_NODES + k,
                              pl.ds(cc * 16, 16)] = accs[cc]
                    return c2

                lax.fori_loop(0, CHUNK_NODES, node_body, 0)
                if half == 1:
                    pair_node = node_base + (ch - 1) * CHUNK_NODES
                    pltpu.async_copy(
                        out_v.at[slot],
                        out_hbm.at[pl.ds(pair_node, 2 * CHUNK_NODES)],
                        sems_o[slot])
                @pl.when(ch + _NBUF_A < n_chunks)
                def _():
                    issue(ch + _NBUF_A, p)
        return carry

    lax.fori_loop(0, MAXG, chunk_group, 0)
    for s in range(_NBUF_A // 2):
        wait_store(s)


@functools.partial(
    pl.kernel,
    out_type=(
        jax.ShapeDtypeStruct((BATCH, N_HIDDEN), jnp.float32),
        jax.ShapeDtypeStruct((BATCH, N_HIDDEN), jnp.float32),
    ),
    mesh=_MESH,
    scratch_types=[
        pltpu.VMEM((BPT,), jnp.int32),
        pltpu.VMEM((BPT,), jnp.int32),
        pltpu.VMEM((BPT, 128), jnp.int32),
        pltpu.VMEM((_NBUF, N_SAMPLE), jnp.int32),
        pltpu.VMEM((_NBUF, N_SAMPLE, N_HIDDEN), jnp.float32),
        pltpu.VMEM((BPT, N_HIDDEN), jnp.float32),
        pltpu.VMEM((BPT, N_HIDDEN), jnp.float32),
        [pltpu.SemaphoreType.DMA] * _NBUF,
        pltpu.SemaphoreType.DMA,
    ],
    compiler_params=pltpu.CompilerParams(needs_layout_passes=False),
)
def _hop2(emb1_hbm, nbr4_hbm, nodes_hbm, self_hbm, agg_hbm,
          nodes_v, rows4_v, nlrows_v, nl32_v, rows_v, self_v, out_v,
          sems, sem_s):
    wid = lax.axis_index("s") * 2 + lax.axis_index("c")
    base = wid * BPT
    pltpu.sync_copy(nodes_hbm.at[pl.ds(base, BPT)], nodes_v)
    for i in range(BPT // 16):
        sl = pl.ds(i * 16, 16)
        rows4_v[sl] = jax.lax.shift_right_logical(nodes_v[sl], 2)
    cp_self = pltpu.async_copy(emb1_hbm.at[nodes_v], self_v, sem_s)
    pltpu.async_copy(nbr4_hbm.at[rows4_v], nlrows_v, sems[0]).wait()

    def issue(k, p):
        kf = jnp.full((16,), k, dtype=jnp.int32)
        nv16 = nodes_v[pl.ds((k >> 4) * 16, 16)]
        node_b = lax.gather(
            nv16, jnp.full((16, 1), k & 15, dtype=jnp.int32),
            lax.GatherDimensionNumbers(offset_dims=(), collapsed_slice_dims=(0,),
                                       start_index_map=(0,)),
            (1,), mode=lax.GatherScatterMode.PROMISE_IN_BOUNDS)
        coloff = (node_b & 3) * N_SAMPLE + lax.iota(jnp.int32, 16)
        nl32_v[p, pl.ds(0, 16)] = plsc.load_gather(nlrows_v, [kf, coloff])
        nl32_v[p, pl.ds(16, 16)] = plsc.load_gather(nlrows_v, [kf, coloff + 16])
        pltpu.async_copy(emb1_hbm.at[nl32_v.at[p]], rows_v.at[p], sems[p])

    def wait(p):
        pltpu.make_async_copy(
            emb1_hbm.at[pl.ds(0, N_SAMPLE)], rows_v.at[p], sems[p]).wait()

    for p in range(_NBUF):
        issue(p, p)

    def seed_group(g, carry):
        for p in range(_NBUF):
            k = g * _NBUF + p
            wait(p)
            accs = _accum32(lambda j, cc: rows_v[p, j, pl.ds(cc * 16, 16)])
            for cc in range(N_HIDDEN // 16):
                out_v[k, pl.ds(cc * 16, 16)] = accs[cc]
            @pl.when(k + _NBUF < BPT)
            def _():
                issue(k + _NBUF, p)
        return carry

    lax.fori_loop(0, BPT // _NBUF, seed_group, 0)
    cp_self.wait()
    pltpu.sync_copy(self_v, self_hbm.at[pl.ds(base, BPT)])
    pltpu.sync_copy(out_v, agg_hbm.at[pl.ds(base, BPT)])


def _l1_body(x_ref, a_ref, w_ref, o_ref):
    y = jnp.dot(x_ref[...], w_ref[0:N_FEATURE, :],
                preferred_element_type=jnp.float32)
    y = y + jnp.dot(a_ref[...] * (1.0 / N_SAMPLE), w_ref[N_FEATURE:, :],
                    preferred_element_type=jnp.float32)
    o_ref[...] = jnp.maximum(y, 0.0)


_L1_BLOCK = 1000


def _layer1(feature, aggsum, W1):
    return pl.pallas_call(
        _l1_body,
        grid=(N_NODES // _L1_BLOCK,),
        in_specs=[
            pl.BlockSpec((_L1_BLOCK, N_FEATURE), lambda i: (i, 0)),
            pl.BlockSpec((_L1_BLOCK, N_FEATURE), lambda i: (i, 0)),
            pl.BlockSpec((2 * N_FEATURE, N_HIDDEN), lambda i: (0, 0)),
        ],
        out_specs=pl.BlockSpec((_L1_BLOCK, N_HIDDEN), lambda i: (i, 0)),
        out_shape=jax.ShapeDtypeStruct((N_NODES, N_HIDDEN), jnp.float32),
    )(feature, aggsum, W1)


def _l2_body(s_ref, a_ref, w2_ref, fw_ref, fb_ref, o_ref):
    x2a = s_ref[...]
    x2b = a_ref[...] * (1.0 / N_SAMPLE)
    h = jnp.dot(x2a, w2_ref[0:N_HIDDEN, :], preferred_element_type=jnp.float32)
    h = h + jnp.dot(x2b, w2_ref[N_HIDDEN:, :],
                    preferred_element_type=jnp.float32)
    h = jnp.maximum(h, 0.0)
    s = jnp.dot(h, fw_ref[...], preferred_element_type=jnp.float32)
    s = s + fb_ref[...]
    m = jnp.max(s, axis=1, keepdims=True)
    lse = jnp.log(jnp.sum(jnp.exp(s - m), axis=1, keepdims=True)) + m
    o_ref[...] = s - lse


def _layer2(selfrows, agg2sum, W2, fc_W, fc_b):
    return pl.pallas_call(
        _l2_body,
        grid=(1,),
        in_specs=[
            pl.BlockSpec((BATCH, N_HIDDEN), lambda i: (0, 0)),
            pl.BlockSpec((BATCH, N_HIDDEN), lambda i: (0, 0)),
            pl.BlockSpec((2 * N_HIDDEN, N_HIDDEN), lambda i: (0, 0)),
            pl.BlockSpec((N_HIDDEN, N_CLASS), lambda i: (0, 0)),
            pl.BlockSpec((1, N_CLASS), lambda i: (0, 0)),
        ],
        out_specs=pl.BlockSpec((BATCH, N_CLASS), lambda i: (0, 0)),
        out_shape=jax.ShapeDtypeStruct((BATCH, N_CLASS), jnp.float32),
    )(selfrows, agg2sum, W2, fc_W, fc_b)


def kernel(nodes, feature, neighbor_list, W1, W2, fc_W, fc_b):
    pad_edges = (NODES_PAD - N_NODES) * N_SAMPLE + IDX_SLACK
    nbr_flat = jnp.concatenate(
        [neighbor_list.reshape(-1),
         jnp.zeros((pad_edges,), dtype=jnp.int32)])
    aggsum = _agg1_sum(feature, nbr_flat)
    emb1 = _layer1(feature, aggsum[:N_NODES], W1)
    nbr4 = neighbor_list.reshape(N_NODES * N_SAMPLE // 128, 128)
    selfrows, agg2sum = _hop2(emb1, nbr4, nodes)
    return _layer2(selfrows, agg2sum, W2, fc_W, fc_b.reshape(1, N_CLASS))
